# baseline (speedup 1.0000x reference)
import numpy as np

import jax
import jax.numpy as jnp
from jax import lax
from jax.experimental import pallas as pl
from jax.experimental.pallas import tpu as pltpu
from jax.experimental.pallas import tpu_sc as plsc

N_NODES = 10000
D = 128
N_EDGES = 320000

NC = 2
NS = 16
NW = NC * NS

BATCH = 128
N_ITERS = 80
E_PER_W = BATCH * N_ITERS
E_PAD = NW * E_PER_W
N_PAD = 10240
ROWS_PER_TILE = N_PAD // NS
N_Q = 4
Q_ITERS = N_ITERS // N_Q


def _sc_aggregate_body(x_hbm, src_hbm, dst_hbm, out_hbm,
                       idx_s, idx_d, rows, agg, sem_g, sem_s, sem_i):
    c = lax.axis_index("c")
    s = lax.axis_index("s")
    wid = s * NC + c

    pltpu.async_copy(src_hbm.at[pl.ds(wid * E_PER_W, E_PER_W)], idx_s, sem_i)
    pltpu.async_copy(dst_hbm.at[wid, 0], idx_d.at[0], sem_i)

    @pl.loop(0, BATCH)
    def _zrow(r):
        for jj in range(D // 16):
            rows[0, r, pl.ds(jj * 16, 16)] = jnp.zeros((16,), jnp.float32)

    r0 = s * ROWS_PER_TILE
    for z in range(ROWS_PER_TILE // BATCH):
        pltpu.sync_copy(rows.at[0], agg.at[pl.ds(r0 + z * BATCH, BATCH)])

    pltpu.make_async_copy(src_hbm.at[pl.ds(0, E_PER_W)], idx_s, sem_i).wait()
    pltpu.make_async_copy(dst_hbm.at[wid, 0], idx_d.at[0], sem_i).wait()
    plsc.subcore_barrier()

    def start_gather(i, b):
        pltpu.async_copy(x_hbm.at[idx_s.at[pl.ds(i * BATCH, BATCH)]],
                         rows.at[b], sem_g)

    def wait_gather(b):
        pltpu.make_async_copy(x_hbm.at[pl.ds(0, BATCH)], rows.at[b], sem_g).wait()

    def wait_scatter(b):
        pltpu.make_async_copy(x_hbm.at[pl.ds(0, BATCH)], rows.at[b], sem_s).wait()

    start_gather(0, 0)
    for q in range(N_Q):
        if q >= 1:
            wait_scatter(1)
        if q + 1 < N_Q:
            pltpu.async_copy(dst_hbm.at[wid, q + 1], idx_d.at[(q + 1) % 2],
                             sem_i)

        @pl.loop(q * Q_ITERS, (q + 1) * Q_ITERS, step=2)
        def _pipe(i0):
            for b in range(2):
                i = i0 + b
                @pl.when(i - q * Q_ITERS >= 1)
                def _():
                    wait_scatter(1 - b)

                @pl.when(i + 1 < N_ITERS)
                def _():
                    start_gather(i + 1, 1 - b)

                wait_gather(b)
                pltpu.async_copy(rows.at[b],
                                 agg.at[idx_d.at[q % 2, i - q * Q_ITERS]],
                                 sem_s, add=True)

        if q + 1 < N_Q:
            pltpu.make_async_copy(dst_hbm.at[wid, 0], idx_d.at[0], sem_i).wait()

    wait_scatter(1)
    plsc.subcore_barrier()

    pltpu.sync_copy(agg.at[pl.ds(r0, ROWS_PER_TILE)],
                    out_hbm.at[pl.ds(c * N_PAD + r0, ROWS_PER_TILE)])


def _sc_aggregate(x, src_flat, dst_q):
    mesh = plsc.VectorSubcoreMesh(core_axis_name="c", subcore_axis_name="s")
    return pl.kernel(
        _sc_aggregate_body,
        out_type=jax.ShapeDtypeStruct((NC * N_PAD, D), jnp.float32),
        mesh=mesh,
        scratch_types=[
            pltpu.VMEM((E_PER_W,), jnp.int32),
            pltpu.VMEM((2, Q_ITERS, BATCH), jnp.int32),
            pltpu.VMEM((2, BATCH, D), jnp.float32),
            pltpu.VMEM_SHARED((N_PAD, D), jnp.float32),
            pltpu.SemaphoreType.DMA,
            pltpu.SemaphoreType.DMA,
            pltpu.SemaphoreType.DMA,
        ],
    )(x, src_flat, dst_q)


def _mlp_body(x_ref, p0_ref, p1_ref, w1_ref, b1_ref, w2_ref, b2_ref, o_ref):
    h = x_ref[...] + p0_ref[...] + p1_ref[...]
    h = jnp.dot(h, w1_ref[...], preferred_element_type=jnp.float32) + b1_ref[...]
    h = jnp.maximum(h, 0.0)
    o_ref[...] = jnp.dot(h, w2_ref[...], preferred_element_type=jnp.float32) + b2_ref[...]


MLP_BLK = 2048
P1_OFF = N_PAD // MLP_BLK


def _mlp(x, partials, W1, b1, W2, b2):
    grid = (pl.cdiv(N_NODES, MLP_BLK),)
    row_spec = pl.BlockSpec((MLP_BLK, D), lambda i: (i, 0))
    p1_spec = pl.BlockSpec((MLP_BLK, D), lambda i: (P1_OFF + i, 0))
    full_spec = pl.BlockSpec((D, D), lambda i: (0, 0))
    bias_spec = pl.BlockSpec((1, D), lambda i: (0, 0))
    return pl.pallas_call(
        _mlp_body,
        grid=grid,
        in_specs=[row_spec, row_spec, p1_spec,
                  full_spec, bias_spec, full_spec, bias_spec],
        out_specs=row_spec,
        out_shape=jax.ShapeDtypeStruct((N_NODES, D), jnp.float32),
    )(x, partials, partials, W1, b1.reshape(1, D), W2, b2.reshape(1, D))


_FILL = np.arange(E_PAD - N_EDGES, dtype=np.int32)
_PAD_EDGES = np.stack([_FILL % N_NODES, N_NODES + _FILL % (N_PAD - N_NODES)])


def kernel(x, edge_index, W1, b1, W2, b2):
    e32 = edge_index.astype(jnp.int32)
    src_flat = jnp.concatenate([e32[0], jnp.asarray(_PAD_EDGES[0])])
    dst_q = jnp.concatenate([e32[1], jnp.asarray(_PAD_EDGES[1])]
                            ).reshape(NW, N_Q, Q_ITERS, BATCH)
    partials = _sc_aggregate(x, src_flat, dst_q)
    return _mlp(x, partials, W1, b1, W2, b2)


# speedup vs baseline: 1.0582x; 1.0582x over previous
import numpy as np

import jax
import jax.numpy as jnp
from jax import lax
from jax.experimental import pallas as pl
from jax.experimental.pallas import tpu as pltpu
from jax.experimental.pallas import tpu_sc as plsc

N_NODES = 10000
D = 128
N_EDGES = 320000

NC = 2
NS = 16
NW = NC * NS

BATCH = 128
N_ITERS = 80
E_PER_W = BATCH * N_ITERS
E_PAD = NW * E_PER_W
N_PAD = 10240
ROWS_PER_TILE = N_PAD // NS
HALF_ITERS = N_ITERS // 2
STEP_BYTES = BATCH * D * 4


def _sc_aggregate_body(x_hbm, e_hbm, out_hbm,
                       idx_s, idx_d, rows, agg, sem_g, sem_s):
    c = lax.axis_index("c")
    s = lax.axis_index("s")
    wid = s * NC + c

    pltpu.async_copy(e_hbm.at[0, wid, pl.ds(0, HALF_ITERS)], idx_s, sem_g)
    pltpu.async_copy(e_hbm.at[1, wid, pl.ds(0, HALF_ITERS)], idx_d, sem_s)

    @pl.loop(0, BATCH)
    def _zrow(r):
        for jj in range(D // 16):
            rows[0, r, pl.ds(jj * 16, 16)] = jnp.zeros((16,), jnp.float32)

    r0 = s * ROWS_PER_TILE
    for z in range(ROWS_PER_TILE // BATCH):
        pltpu.sync_copy(rows.at[0], agg.at[pl.ds(r0 + z * BATCH, BATCH)])

    pltpu.make_async_copy(e_hbm.at[0, wid, pl.ds(0, HALF_ITERS)], idx_s, sem_g).wait()
    pltpu.make_async_copy(e_hbm.at[1, wid, pl.ds(0, HALF_ITERS)], idx_d, sem_s).wait()
    plsc.subcore_barrier()

    def start_gather(i, b):
        pltpu.async_copy(x_hbm.at[idx_s.at[i]], rows.at[b], sem_g)

    def start_scatter(i, b):
        pltpu.async_copy(rows.at[b], agg.at[idx_d.at[i]], sem_s, add=True)

    def wait_gather(b):
        pltpu.make_async_copy(x_hbm.at[pl.ds(0, BATCH)], rows.at[b], sem_g).wait()

    def wait_scatter(b):
        pltpu.make_async_copy(x_hbm.at[pl.ds(0, BATCH)], rows.at[b], sem_s).wait()

    for h in range(2):
        if h > 0:
            pltpu.sync_copy(e_hbm.at[0, wid, pl.ds(h * HALF_ITERS, HALF_ITERS)], idx_s)
            pltpu.sync_copy(e_hbm.at[1, wid, pl.ds(h * HALF_ITERS, HALF_ITERS)], idx_d)
        start_gather(0, 0)

        @pl.loop(0, HALF_ITERS, step=2)
        def _pipe(i0):
            for b in range(2):
                i = i0 + b
                @pl.when(i >= 1)
                def _():
                    wait_scatter(1 - b)

                @pl.when(i + 1 < HALF_ITERS)
                def _():
                    start_gather(i + 1, 1 - b)

                wait_gather(b)
                start_scatter(i, b)

        wait_scatter(1)

    plsc.subcore_barrier()

    pltpu.sync_copy(agg.at[pl.ds(r0, ROWS_PER_TILE)],
                    out_hbm.at[pl.ds(c * N_PAD + r0, ROWS_PER_TILE)])


def _sc_aggregate(x, edges):
    mesh = plsc.VectorSubcoreMesh(core_axis_name="c", subcore_axis_name="s")
    return pl.kernel(
        _sc_aggregate_body,
        out_type=jax.ShapeDtypeStruct((NC * N_PAD, D), jnp.float32),
        mesh=mesh,
        scratch_types=[
            pltpu.VMEM((HALF_ITERS, BATCH), jnp.int32),
            pltpu.VMEM((HALF_ITERS, BATCH), jnp.int32),
            pltpu.VMEM((2, BATCH, D), jnp.float32),
            pltpu.VMEM_SHARED((N_PAD, D), jnp.float32),
            pltpu.SemaphoreType.DMA,
            pltpu.SemaphoreType.DMA,
        ],
    )(x, edges)


def _mlp_body(x_ref, p0_ref, p1_ref, w1_ref, b1_ref, w2_ref, b2_ref, o_ref):
    h = x_ref[...] + p0_ref[...] + p1_ref[...]
    h = jnp.dot(h, w1_ref[...], preferred_element_type=jnp.float32) + b1_ref[...]
    h = jnp.maximum(h, 0.0)
    o_ref[...] = jnp.dot(h, w2_ref[...], preferred_element_type=jnp.float32) + b2_ref[...]


MLP_BLK = 2048
P1_OFF = N_PAD // MLP_BLK


def _mlp(x, partials, W1, b1, W2, b2):
    grid = (pl.cdiv(N_NODES, MLP_BLK),)
    row_spec = pl.BlockSpec((MLP_BLK, D), lambda i: (i, 0))
    p1_spec = pl.BlockSpec((MLP_BLK, D), lambda i: (P1_OFF + i, 0))
    full_spec = pl.BlockSpec((D, D), lambda i: (0, 0))
    bias_spec = pl.BlockSpec((1, D), lambda i: (0, 0))
    return pl.pallas_call(
        _mlp_body,
        grid=grid,
        in_specs=[row_spec, row_spec, p1_spec,
                  full_spec, bias_spec, full_spec, bias_spec],
        out_specs=row_spec,
        out_shape=jax.ShapeDtypeStruct((N_NODES, D), jnp.float32),
    )(x, partials, partials, W1, b1.reshape(1, D), W2, b2.reshape(1, D))


_FILL = np.arange(E_PAD - N_EDGES, dtype=np.int32)
_PAD_EDGES = np.stack([_FILL % N_NODES, N_NODES + _FILL % (N_PAD - N_NODES)])


def kernel(x, edge_index, W1, b1, W2, b2):
    edges = jnp.concatenate([edge_index.astype(jnp.int32),
                             jnp.asarray(_PAD_EDGES)], axis=1)
    edges = edges.reshape(2, NW, N_ITERS, BATCH)
    partials = _sc_aggregate(x, edges)
    return _mlp(x, partials, W1, b1, W2, b2)


# speedup vs baseline: 1.0630x; 1.0046x over previous
import numpy as np

import jax
import jax.numpy as jnp
from jax import lax
from jax.experimental import pallas as pl
from jax.experimental.pallas import tpu as pltpu
from jax.experimental.pallas import tpu_sc as plsc

N_NODES = 10000
D = 128
N_EDGES = 320000

NC = 2
NS = 16
NW = NC * NS

BATCH = 128
N_ITERS = 80
E_PER_W = BATCH * N_ITERS
E_PAD = NW * E_PER_W
N_PAD = 10240
ROWS_PER_TILE = N_PAD // NS
HALF_ITERS = N_ITERS // 2
STEP_BYTES = BATCH * D * 4


def _sc_aggregate_body(x_hbm, e_hbm, out_hbm,
                       idx_s, idx_d, rows, agg, sem_g, sem_s):
    c = lax.axis_index("c")
    s = lax.axis_index("s")
    wid = s * NC + c

    pltpu.async_copy(e_hbm.at[0, wid, pl.ds(0, HALF_ITERS)], idx_s, sem_g)
    pltpu.async_copy(e_hbm.at[1, wid, pl.ds(0, HALF_ITERS)], idx_d, sem_s)

    @pl.loop(0, BATCH)
    def _zrow(r):
        for jj in range(D // 16):
            rows[0, r, pl.ds(jj * 16, 16)] = jnp.zeros((16,), jnp.float32)

    r0 = s * ROWS_PER_TILE
    for z in range(ROWS_PER_TILE // BATCH):
        pltpu.sync_copy(rows.at[0], agg.at[pl.ds(r0 + z * BATCH, BATCH)])

    pltpu.make_async_copy(e_hbm.at[0, wid, pl.ds(0, HALF_ITERS)], idx_s, sem_g).wait()
    pltpu.make_async_copy(e_hbm.at[1, wid, pl.ds(0, HALF_ITERS)], idx_d, sem_s).wait()
    plsc.subcore_barrier()

    def start_gather(i, b):
        pltpu.async_copy(x_hbm.at[idx_s.at[i]], rows.at[b], sem_g)

    def start_scatter(i, b):
        pltpu.async_copy(rows.at[b], agg.at[idx_d.at[i]], sem_s, add=True)

    def wait_gather(b):
        pltpu.make_async_copy(x_hbm.at[pl.ds(0, BATCH)], rows.at[b], sem_g).wait()

    def wait_scatter(b):
        pltpu.make_async_copy(x_hbm.at[pl.ds(0, BATCH)], rows.at[b], sem_s).wait()

    for h in range(2):
        if h > 0:
            pltpu.sync_copy(e_hbm.at[0, wid, pl.ds(h * HALF_ITERS, HALF_ITERS)], idx_s)
            pltpu.sync_copy(e_hbm.at[1, wid, pl.ds(h * HALF_ITERS, HALF_ITERS)], idx_d)
        start_gather(0, 0)

        @pl.loop(0, HALF_ITERS, step=2)
        def _pipe(i0):
            for b in range(2):
                i = i0 + b
                @pl.when(i >= 1)
                def _():
                    wait_scatter(1 - b)

                @pl.when(i + 1 < HALF_ITERS)
                def _():
                    start_gather(i + 1, 1 - b)

                wait_gather(b)
                start_scatter(i, b)

        wait_scatter(1)

    plsc.subcore_barrier()

    pltpu.sync_copy(agg.at[pl.ds(r0, ROWS_PER_TILE)],
                    out_hbm.at[pl.ds(c * N_PAD + r0, ROWS_PER_TILE)])


def _sc_aggregate(x, edges):
    mesh = plsc.VectorSubcoreMesh(core_axis_name="c", subcore_axis_name="s")
    return pl.kernel(
        _sc_aggregate_body,
        out_type=jax.ShapeDtypeStruct((NC * N_PAD, D), jnp.float32),
        mesh=mesh,
        scratch_types=[
            pltpu.VMEM((HALF_ITERS, BATCH), jnp.int32),
            pltpu.VMEM((HALF_ITERS, BATCH), jnp.int32),
            pltpu.VMEM((2, BATCH, D), jnp.float32),
            pltpu.VMEM_SHARED((N_PAD, D), jnp.float32),
            pltpu.SemaphoreType.DMA,
            pltpu.SemaphoreType.DMA,
        ],
    )(x, edges)


def _mlp_body(x_ref, p_ref, w1_ref, b1_ref, w2_ref, b2_ref, o_ref):
    h = x_ref[...] + p_ref[0] + p_ref[1]
    h = jnp.dot(h, w1_ref[...], preferred_element_type=jnp.float32) + b1_ref[...]
    h = jnp.maximum(h, 0.0)
    o_ref[...] = jnp.dot(h, w2_ref[...], preferred_element_type=jnp.float32) + b2_ref[...]


MLP_BLK = 2048


def _mlp(x, partials, W1, b1, W2, b2):
    grid = (pl.cdiv(N_NODES, MLP_BLK),)
    row_spec = pl.BlockSpec((MLP_BLK, D), lambda i: (i, 0))
    p_spec = pl.BlockSpec((2, MLP_BLK, D), lambda i: (0, i, 0))
    full_spec = pl.BlockSpec((D, D), lambda i: (0, 0))
    bias_spec = pl.BlockSpec((1, D), lambda i: (0, 0))
    return pl.pallas_call(
        _mlp_body,
        grid=grid,
        in_specs=[row_spec, p_spec,
                  full_spec, bias_spec, full_spec, bias_spec],
        out_specs=row_spec,
        out_shape=jax.ShapeDtypeStruct((N_NODES, D), jnp.float32),
    )(x, partials.reshape(NC, N_PAD, D), W1, b1.reshape(1, D), W2,
      b2.reshape(1, D))


_FILL = np.arange(E_PAD - N_EDGES, dtype=np.int32)
_PAD_EDGES = np.stack([_FILL % N_NODES, N_NODES + _FILL % (N_PAD - N_NODES)])


def kernel(x, edge_index, W1, b1, W2, b2):
    edges = jnp.concatenate([edge_index.astype(jnp.int32),
                             jnp.asarray(_PAD_EDGES)], axis=1)
    edges = edges.reshape(2, NW, N_ITERS, BATCH)
    partials = _sc_aggregate(x, edges)
    return _mlp(x, partials, W1, b1, W2, b2)
